# baseline (speedup 1.0000x reference)
import functools

import jax
import jax.numpy as jnp
from jax import lax
from jax.experimental import pallas as pl
from jax.experimental.pallas import tpu as pltpu
from jax.experimental.pallas import tpu_sc as plsc

NC, NS, LANES = 2, 16, 16
NW = NC * NS
B, L, D = 4096, 200, 64
N = B * L
C = 128
CPW = N // (NW * C)
VPR = D // LANES


def _sc_body(x_hbm, table_hbm, pos_hbm, out_hbm, idx_v, pos_v, rows_v, gsem):
    wid = lax.axis_index("s") * NC + lax.axis_index("c")
    pltpu.sync_copy(x_hbm.at[pl.ds(wid * CPW, CPW)], idx_v)
    pltpu.sync_copy(pos_hbm, pos_v)

    def chunk(c, _):
        g = wid * CPW + c
        pltpu.async_copy(table_hbm.at[idx_v.at[c]], rows_v, gsem).wait()
        base = lax.rem(c * C, L)

        def add_row(j, _):
            p = base + j
            for q in range(VPR):
                s = pl.ds(q * LANES, LANES)
                rows_v[j, s] = rows_v[j, s] + pos_v[p, s]
            return 0

        lax.fori_loop(0, C, add_row, 0, unroll=4)
        pltpu.sync_copy(rows_v, out_hbm.at[pl.ds(g * C, C)])
        return 0

    lax.fori_loop(0, CPW, chunk, 0)


@jax.jit
def _embed(x_flat2d, token_table, pos_dup):
    mesh = plsc.VectorSubcoreMesh(core_axis_name="c", subcore_axis_name="s")
    return pl.kernel(
        _sc_body,
        out_type=jax.ShapeDtypeStruct((N, D), jnp.float32),
        mesh=mesh,
        scratch_types=[
            pltpu.VMEM((CPW, C), jnp.int32),
            pltpu.VMEM((2 * L, D), jnp.float32),
            pltpu.VMEM((C, D), jnp.float32),
            pltpu.SemaphoreType.DMA,
        ],
        compiler_params=pltpu.CompilerParams(use_tc_tiling_on_sc=False),
    )(x_flat2d, token_table, pos_dup)


def kernel(x, token_table, time_embedding):
    x_flat2d = x.reshape(NW * CPW, C)
    pos = jnp.transpose(time_embedding)
    pos_dup = jnp.concatenate([pos, pos], axis=0)
    out = _embed(x_flat2d, token_table, pos_dup)
    return out.reshape(B, L, D)


# speedup vs baseline: 1.1794x; 1.1794x over previous
import functools

import jax
import jax.numpy as jnp
from jax import lax
from jax.experimental import pallas as pl
from jax.experimental.pallas import tpu as pltpu
from jax.experimental.pallas import tpu_sc as plsc

NC, NS, LANES = 2, 16, 16
NW = NC * NS
B, L, D = 4096, 200, 64
N = B * L
C = 128
CPW = N // (NW * C)
VPR = D // LANES
NBUF = 8
LOOKAHEAD = 4
ROUNDS = CPW // NBUF


def _sc_body(x_hbm, table_hbm, pos_hbm, out_hbm, idx_v, pos_v, bufs, gsems, ssems):
    wid = lax.axis_index("s") * NC + lax.axis_index("c")
    pltpu.sync_copy(x_hbm.at[pl.ds(wid * CPW, CPW)], idx_v)
    pltpu.sync_copy(pos_hbm, pos_v)
    out_base = wid * CPW * C

    def gather(c, b):
        return pltpu.make_async_copy(table_hbm.at[idx_v.at[c]], bufs[b], gsems[b])

    def store(c, b):
        return pltpu.make_async_copy(
            bufs[b], out_hbm.at[pl.ds(out_base + c * C, C)], ssems[b])

    for b in range(LOOKAHEAD):
        gather(b, b).start()

    def round_body(r, _):
        for b in range(NBUF):
            c = r * NBUF + b
            gather(c, b).wait()
            base = lax.rem(c * C, L)

            def add_row(j, _):
                p = base + j
                for q in range(VPR):
                    s = pl.ds(q * LANES, LANES)
                    bufs[b][j, s] = bufs[b][j, s] + pos_v[p, s]
                return 0

            lax.fori_loop(0, C, add_row, 0, unroll=4)
            store(c, b).start()

            f = c + LOOKAHEAD
            bf = (b + LOOKAHEAD) % NBUF

            @pl.when(f < CPW)
            def _():
                @pl.when(f >= NBUF)
                def _():
                    store(f - NBUF, bf).wait()
                gather(f, bf).start()
        return 0

    lax.fori_loop(0, ROUNDS, round_body, 0)
    for b in range(NBUF):
        store(CPW - NBUF + b, b).wait()


@jax.jit
def _embed(x_flat2d, token_table, pos_dup):
    mesh = plsc.VectorSubcoreMesh(core_axis_name="c", subcore_axis_name="s")
    return pl.kernel(
        _sc_body,
        out_type=jax.ShapeDtypeStruct((N, D), jnp.float32),
        mesh=mesh,
        scratch_types=[
            pltpu.VMEM((CPW, C), jnp.int32),
            pltpu.VMEM((2 * L, D), jnp.float32),
            [pltpu.VMEM((C, D), jnp.float32) for _ in range(NBUF)],
            [pltpu.SemaphoreType.DMA for _ in range(NBUF)],
            [pltpu.SemaphoreType.DMA for _ in range(NBUF)],
        ],
        compiler_params=pltpu.CompilerParams(use_tc_tiling_on_sc=False),
    )(x_flat2d, token_table, pos_dup)


def kernel(x, token_table, time_embedding):
    x_flat2d = x.reshape(NW * CPW, C)
    pos = jnp.transpose(time_embedding)
    pos_dup = jnp.concatenate([pos, pos], axis=0)
    out = _embed(x_flat2d, token_table, pos_dup)
    return out.reshape(B, L, D)
